# Initial kernel scaffold; baseline (speedup 1.0000x reference)
import jax
import jax.numpy as jnp
from jax.experimental import pallas as pl


def kernel(x, edge_index, edge_attr, W_l, W_r, att, bias, gamma, beta):
    raise NotImplementedError("write your pallas kernel here")


# speedup vs baseline: 1.1984x; 1.1984x over previous
import functools

import jax
import jax.numpy as jnp
from jax.experimental import pallas as pl
from jax.experimental.pallas import tpu as pltpu


def _mm_body(x_ref, w_ref, o_ref):
    o_ref[...] = jnp.dot(x_ref[...], w_ref[...],
                         preferred_element_type=jnp.float32)


def _matmul(x, w):
    M, K = x.shape
    N = w.shape[1]
    BM = 1000
    return pl.pallas_call(
        _mm_body,
        grid=(M // BM,),
        in_specs=[pl.BlockSpec((BM, K), lambda i: (i, 0)),
                  pl.BlockSpec((K, N), lambda i: (0, 0))],
        out_specs=pl.BlockSpec((BM, N), lambda i: (i, 0)),
        out_shape=jax.ShapeDtypeStruct((M, N), jnp.float32),
    )(x, w)


def _bn_body(acc_ref, den_ref, bias_ref, gamma_ref, beta_ref, o_ref):
    v = acc_ref[...] / (den_ref[...] + 1e-16) + bias_ref[...]
    n = v.shape[0]
    mean = jnp.sum(v, axis=0, keepdims=True) / n
    d = v - mean
    var = jnp.sum(d * d, axis=0, keepdims=True) / n
    o_ref[...] = gamma_ref[...] * d * jax.lax.rsqrt(var + 1e-5) + beta_ref[...]


def _bn_epilogue(acc, denom, bias, gamma, beta):
    N, D = acc.shape
    return pl.pallas_call(
        _bn_body,
        in_specs=[pl.BlockSpec((N, D), lambda: (0, 0)),
                  pl.BlockSpec((N, 1), lambda: (0, 0)),
                  pl.BlockSpec((1, D), lambda: (0, 0)),
                  pl.BlockSpec((1, D), lambda: (0, 0)),
                  pl.BlockSpec((1, D), lambda: (0, 0))],
        out_specs=pl.BlockSpec((N, D), lambda: (0, 0)),
        out_shape=jax.ShapeDtypeStruct((N, D), jnp.float32),
    )(acc, denom.reshape(N, 1), bias.reshape(1, D),
      gamma.reshape(1, D), beta.reshape(1, D))


def kernel(x, edge_index, edge_attr, W_l, W_r, att, bias, gamma, beta):
    N, D = x.shape
    W = jnp.concatenate([W_l, W_r], axis=1)
    xw = _matmul(x, W)
    xl = xw[:, :D]
    xr = xw[:, D:]

    loops = jnp.arange(N, dtype=edge_index.dtype)
    src = jnp.concatenate([edge_index[0], loops])
    dst = jnp.concatenate([edge_index[1], loops])

    h = jax.nn.leaky_relu(xl[src] + xr[dst], 0.2)
    e = h @ att
    m = jax.ops.segment_max(e, dst, num_segments=N)
    p = jnp.exp(e - m[dst])
    denom = jax.ops.segment_sum(p, dst, num_segments=N)
    acc = jax.ops.segment_sum(p[:, None] * xl[src], dst, num_segments=N)

    out = _bn_epilogue(acc, denom, bias, gamma, beta)
    return (out, edge_index, edge_attr)


# speedup vs baseline: 3.4326x; 2.8643x over previous
import functools

import jax
import jax.numpy as jnp
from jax import lax
from jax.experimental import pallas as pl
from jax.experimental.pallas import tpu as pltpu
from jax.experimental.pallas import tpu_sc as plsc

_NC = 2
_NS = 16
_L = 16
_NW = _NC * _NS


def _mm_body(x_ref, w_ref, o_ref):
    o_ref[...] = jnp.dot(x_ref[...], w_ref[...],
                         preferred_element_type=jnp.float32)


def _matmul(x, w):
    M, K = x.shape
    N = w.shape[1]
    BM = 1000
    return pl.pallas_call(
        _mm_body,
        grid=(M // BM,),
        in_specs=[pl.BlockSpec((BM, K), lambda i: (i, 0)),
                  pl.BlockSpec((K, N), lambda i: (0, 0))],
        out_specs=pl.BlockSpec((BM, N), lambda i: (i, 0)),
        out_shape=jax.ShapeDtypeStruct((M, N), jnp.float32),
    )(x, w)


def _bn_body(acc_ref, den_ref, bias_ref, gamma_ref, beta_ref, o_ref):
    v = acc_ref[...] / (den_ref[...] + 1e-16) + bias_ref[...]
    n = v.shape[0]
    mean = jnp.sum(v, axis=0, keepdims=True) / n
    d = v - mean
    var = jnp.sum(d * d, axis=0, keepdims=True) / n
    o_ref[...] = gamma_ref[...] * d * jax.lax.rsqrt(var + 1e-5) + beta_ref[...]


def _bn_epilogue(acc, denom, bias, gamma, beta):
    N, D = acc.shape
    return pl.pallas_call(
        _bn_body,
        in_specs=[pl.BlockSpec((N, D), lambda: (0, 0)),
                  pl.BlockSpec((N, 1), lambda: (0, 0)),
                  pl.BlockSpec((1, D), lambda: (0, 0)),
                  pl.BlockSpec((1, D), lambda: (0, 0)),
                  pl.BlockSpec((1, D), lambda: (0, 0))],
        out_specs=pl.BlockSpec((N, D), lambda: (0, 0)),
        out_shape=jax.ShapeDtypeStruct((N, D), jnp.float32),
    )(acc, denom.reshape(N, 1), bias.reshape(1, D),
      gamma.reshape(1, D), beta.reshape(1, D))


def _edge_scores(xl, xr, src, dst, att, e_real):
    D = xl.shape[1]
    e_pad = src.shape[0]
    ept = e_pad // _NW
    nch = ept // _L
    mesh = plsc.VectorSubcoreMesh(core_axis_name="c", subcore_axis_name="s")

    @functools.partial(
        pl.kernel,
        out_type=jax.ShapeDtypeStruct((e_pad,), jnp.float32),
        mesh=mesh,
        compiler_params=pltpu.CompilerParams(use_tc_tiling_on_sc=False),
        scratch_types=[
            pltpu.VMEM((ept,), jnp.int32),
            pltpu.VMEM((ept,), jnp.int32),
            pltpu.VMEM((D,), jnp.float32),
            pltpu.VMEM((_L, D), jnp.float32),
            pltpu.VMEM((_L, D), jnp.float32),
            pltpu.VMEM((ept,), jnp.float32),
            pltpu.SemaphoreType.DMA,
            pltpu.SemaphoreType.DMA,
        ],
    )
    def k(xl_hbm, xr_hbm, src_hbm, dst_hbm, att_hbm, p_hbm,
          src_v, dst_v, att_v, rows_l, rows_r, p_buf, sem_l, sem_r):
        wid = lax.axis_index("s") * _NC + lax.axis_index("c")
        base = wid * ept
        pltpu.sync_copy(src_hbm.at[pl.ds(base, ept)], src_v)
        pltpu.sync_copy(dst_hbm.at[pl.ds(base, ept)], dst_v)
        pltpu.sync_copy(att_hbm, att_v)
        iota = lax.iota(jnp.int32, _L)
        shufs = [jnp.bitwise_xor(iota, sh) for sh in (8, 4, 2, 1)]
        att_vecs = [att_v[pl.ds(c * _L, _L)] for c in range(D // _L)]

        def body(i, carry):
            o = i * _L
            sidx = src_v[pl.ds(o, _L)]
            didx = dst_v[pl.ds(o, _L)]
            cl = pltpu.async_copy(xl_hbm.at[sidx], rows_l, sem_l)
            cr = pltpu.async_copy(xr_hbm.at[didx], rows_r, sem_r)
            cl.wait()
            cr.wait()
            e = jnp.zeros((_L,), jnp.float32)
            for j in range(_L):
                acc = jnp.zeros((_L,), jnp.float32)
                for c in range(D // _L):
                    sl = pl.ds(c * _L, _L)
                    v = rows_l[j, sl] + rows_r[j, sl]
                    acc = acc + att_vecs[c] * jnp.where(v >= 0, v, 0.2 * v)
                for sf in shufs:
                    acc = acc + acc[sf]
                e = jnp.where(iota == j, acc, e)
            p = jnp.exp(e)
            eid = iota + jnp.full((_L,), base + o, jnp.int32)
            p = jnp.where(eid < e_real, p, 0.0)
            p_buf[pl.ds(o, _L)] = p
            return carry

        lax.fori_loop(0, nch, body, 0)
        pltpu.sync_copy(p_buf, p_hbm.at[pl.ds(base, ept)])

    return k(xl, xr, src, dst, att)


def _scatter_pass(xlab, src, dst, p, zrows, n_nodes, n_rows, hw):
    W = hw + _L
    e_pad = src.shape[0]
    ept = e_pad // _NS
    nch = ept // _L
    rpt = n_rows // _NS
    mesh = plsc.VectorSubcoreMesh(core_axis_name="c", subcore_axis_name="s")

    @functools.partial(
        pl.kernel,
        out_type=jax.ShapeDtypeStruct((_NC, n_rows, W), jnp.float32),
        mesh=mesh,
        compiler_params=pltpu.CompilerParams(use_tc_tiling_on_sc=False),
        scratch_types=[
            pltpu.VMEM((ept,), jnp.int32),
            pltpu.VMEM((ept,), jnp.int32),
            pltpu.VMEM((ept,), jnp.float32),
            pltpu.VMEM((_L, hw), jnp.float32),
            pltpu.VMEM((_L, W), jnp.float32),
            pltpu.VMEM_SHARED((n_rows, W), jnp.float32),
            pltpu.SemaphoreType.DMA,
        ],
    )
    def k(xlab_hbm, src_hbm, dst_hbm, p_hbm, z_hbm, out_hbm,
          src_v, dst_v, p_v, rows_g, rows_s, acc_sh, sem):
        cid = lax.axis_index("c")
        sid = lax.axis_index("s")
        base = sid * ept
        pltpu.sync_copy(src_hbm.at[pl.ds(base, ept)], src_v)
        pltpu.sync_copy(dst_hbm.at[pl.ds(base, ept)], dst_v)
        pltpu.sync_copy(p_hbm.at[pl.ds(base, ept)], p_v)
        pltpu.sync_copy(z_hbm, acc_sh.at[pl.ds(sid * rpt, rpt)])
        plsc.subcore_barrier()
        iota = lax.iota(jnp.int32, _L)
        onehot = jnp.where(iota == 0, jnp.float32(1.0), jnp.float32(0.0))
        off = cid * n_nodes

        def body(i, carry):
            o = i * _L
            sidx = src_v[pl.ds(o, _L)] + jnp.full((_L,), off, jnp.int32)
            didx = dst_v[pl.ds(o, _L)]
            pltpu.async_copy(xlab_hbm.at[sidx], rows_g, sem).wait()
            pvec = p_v[pl.ds(o, _L)]
            for j in range(_L):
                pj = jnp.full((_L,), pvec[j], jnp.float32)
                for cc in range(hw // _L):
                    sl = pl.ds(cc * _L, _L)
                    rows_s[j, sl] = rows_g[j, sl] * pj
                rows_s[j, pl.ds(hw, _L)] = pj * onehot
            pltpu.sync_copy(rows_s, acc_sh.at[didx], add=True)
            return carry

        lax.fori_loop(0, nch, body, 0)
        plsc.subcore_barrier()
        pltpu.sync_copy(acc_sh.at[pl.ds(sid * rpt, rpt)],
                        out_hbm.at[cid, pl.ds(sid * rpt, rpt)])

    return k(xlab, src, dst, p, zrows)


def kernel(x, edge_index, edge_attr, W_l, W_r, att, bias, gamma, beta):
    N, D = x.shape
    E = edge_index.shape[1]
    idt = edge_index.dtype

    W = jnp.concatenate([W_l, W_r], axis=1)
    xw = _matmul(x, W)
    xl = xw[:, :D]
    xr = xw[:, D:]

    e_real = E + N
    e_pad = ((e_real + _NW * _L - 1) // (_NW * _L)) * (_NW * _L)
    loops = jnp.arange(N, dtype=idt)
    padz = jnp.zeros((e_pad - e_real,), dtype=idt)
    src = jnp.concatenate([edge_index[0], loops, padz]).astype(jnp.int32)
    dst = jnp.concatenate([edge_index[1], loops, padz]).astype(jnp.int32)

    p = _edge_scores(xl, xr, src, dst, att, e_real)

    hw = D // 2
    xlab = jnp.concatenate([xl[:, :hw], xl[:, hw:]], axis=0)
    n_rows = ((N + _NS * _L - 1) // (_NS * _L)) * (_NS * _L)
    zrows = jnp.zeros((n_rows // _NS, hw + _L), jnp.float32)
    accs = _scatter_pass(xlab, src, dst, p, zrows, N, n_rows, hw)

    acc = jnp.concatenate([accs[0, :N, :hw], accs[1, :N, :hw]], axis=1)
    denom = accs[0, :N, hw]

    out = _bn_epilogue(acc, denom, bias, gamma, beta)
    return (out, edge_index, edge_attr)


# speedup vs baseline: 4.9605x; 1.4451x over previous
import functools

import jax
import jax.numpy as jnp
from jax import lax
from jax.experimental import pallas as pl
from jax.experimental.pallas import tpu as pltpu
from jax.experimental.pallas import tpu_sc as plsc

_NC = 2
_NS = 16
_L = 16
_NW = _NC * _NS


def _mm_body(x_ref, w_ref, o_ref):
    o_ref[...] = jnp.dot(x_ref[...], w_ref[...],
                         preferred_element_type=jnp.float32)


def _matmul(x, w):
    M, K = x.shape
    N = w.shape[1]
    BM = 1000
    return pl.pallas_call(
        _mm_body,
        grid=(M // BM,),
        in_specs=[pl.BlockSpec((BM, K), lambda i: (i, 0)),
                  pl.BlockSpec((K, N), lambda i: (0, 0))],
        out_specs=pl.BlockSpec((BM, N), lambda i: (i, 0)),
        out_shape=jax.ShapeDtypeStruct((M, N), jnp.float32),
    )(x, w)


def _bn_body(acc_ref, den_ref, bias_ref, gamma_ref, beta_ref, o_ref):
    v = acc_ref[...] / (den_ref[...] + 1e-16) + bias_ref[...]
    n = v.shape[0]
    mean = jnp.sum(v, axis=0, keepdims=True) / n
    d = v - mean
    var = jnp.sum(d * d, axis=0, keepdims=True) / n
    o_ref[...] = gamma_ref[...] * d * jax.lax.rsqrt(var + 1e-5) + beta_ref[...]


def _bn_epilogue(acc, denom, bias, gamma, beta):
    N, D = acc.shape
    return pl.pallas_call(
        _bn_body,
        in_specs=[pl.BlockSpec((N, D), lambda: (0, 0)),
                  pl.BlockSpec((N, 1), lambda: (0, 0)),
                  pl.BlockSpec((1, D), lambda: (0, 0)),
                  pl.BlockSpec((1, D), lambda: (0, 0)),
                  pl.BlockSpec((1, D), lambda: (0, 0))],
        out_specs=pl.BlockSpec((N, D), lambda: (0, 0)),
        out_shape=jax.ShapeDtypeStruct((N, D), jnp.float32),
    )(acc, denom.reshape(N, 1), bias.reshape(1, D),
      gamma.reshape(1, D), beta.reshape(1, D))


def _edge_scores(xl, xr, src, dst, att, e_real):
    D = xl.shape[1]
    e_pad = src.shape[0]
    ept = e_pad // _NW
    nch = ept // _L
    npair = nch // 2
    mesh = plsc.VectorSubcoreMesh(core_axis_name="c", subcore_axis_name="s")

    @functools.partial(
        pl.kernel,
        out_type=jax.ShapeDtypeStruct((e_pad,), jnp.float32),
        mesh=mesh,
        compiler_params=pltpu.CompilerParams(use_tc_tiling_on_sc=False),
        scratch_types=[
            pltpu.VMEM((ept,), jnp.int32),
            pltpu.VMEM((ept,), jnp.int32),
            pltpu.VMEM((D,), jnp.float32),
            pltpu.VMEM((2, _L, D), jnp.float32),
            pltpu.VMEM((2, _L, D), jnp.float32),
            pltpu.VMEM((ept,), jnp.float32),
            pltpu.SemaphoreType.DMA,
            pltpu.SemaphoreType.DMA,
            pltpu.SemaphoreType.DMA,
            pltpu.SemaphoreType.DMA,
        ],
    )
    def k(xl_hbm, xr_hbm, src_hbm, dst_hbm, att_hbm, p_hbm,
          src_v, dst_v, att_v, rows_l, rows_r, p_buf,
          sem_l0, sem_l1, sem_r0, sem_r1):
        wid = lax.axis_index("s") * _NC + lax.axis_index("c")
        base = wid * ept
        pltpu.sync_copy(src_hbm.at[pl.ds(base, ept)], src_v)
        pltpu.sync_copy(dst_hbm.at[pl.ds(base, ept)], dst_v)
        pltpu.sync_copy(att_hbm, att_v)
        iota = lax.iota(jnp.int32, _L)
        shufs = [jnp.bitwise_xor(iota, sh) for sh in (8, 4, 2, 1)]
        att_vecs = [att_v[pl.ds(c * _L, _L)] for c in range(D // _L)]
        sems = [(sem_l0, sem_r0), (sem_l1, sem_r1)]

        def fire(c, b):
            o = c * _L
            sl, sr = sems[b]
            pltpu.async_copy(xl_hbm.at[src_v[pl.ds(o, _L)]], rows_l.at[b], sl)
            pltpu.async_copy(xr_hbm.at[dst_v[pl.ds(o, _L)]], rows_r.at[b], sr)

        def wait(b):
            sl, sr = sems[b]
            pltpu.make_async_copy(xl_hbm.at[src_v[pl.ds(0, _L)]],
                                  rows_l.at[b], sl).wait()
            pltpu.make_async_copy(xr_hbm.at[dst_v[pl.ds(0, _L)]],
                                  rows_r.at[b], sr).wait()

        def compute(c, b):
            o = c * _L
            e = jnp.zeros((_L,), jnp.float32)
            for j in range(_L):
                acc = jnp.zeros((_L,), jnp.float32)
                for cc in range(D // _L):
                    sl = pl.ds(cc * _L, _L)
                    v = rows_l[b, j, sl] + rows_r[b, j, sl]
                    acc = acc + att_vecs[cc] * jnp.where(v >= 0, v, 0.2 * v)
                for sf in shufs:
                    acc = acc + acc[sf]
                e = jnp.where(iota == j, acc, e)
            p = jnp.exp(e)
            eid = iota + jnp.full((_L,), base + o, jnp.int32)
            p = jnp.where(eid < e_real, p, 0.0)
            p_buf[pl.ds(o, _L)] = p

        fire(0, 0)

        def body(k_, carry):
            c0 = k_ * 2
            fire(c0 + 1, 1)
            wait(0)
            compute(c0, 0)

            @pl.when(k_ < npair - 1)
            def _():
                fire(c0 + 2, 0)

            wait(1)
            compute(c0 + 1, 1)
            return carry

        lax.fori_loop(0, npair, body, 0)
        pltpu.sync_copy(p_buf, p_hbm.at[pl.ds(base, ept)])

    return k(xl, xr, src, dst, att)


def _scatter_pass(xlab, src, dst, p, zrows, n_nodes, n_rows, hw):
    W = hw + _L
    e_pad = src.shape[0]
    ept = e_pad // _NS
    nch = ept // _L
    rpt = n_rows // _NS
    mesh = plsc.VectorSubcoreMesh(core_axis_name="c", subcore_axis_name="s")

    npair = nch // 2

    @functools.partial(
        pl.kernel,
        out_type=jax.ShapeDtypeStruct((_NC, n_rows, W), jnp.float32),
        mesh=mesh,
        compiler_params=pltpu.CompilerParams(use_tc_tiling_on_sc=False),
        scratch_types=[
            pltpu.VMEM((ept,), jnp.int32),
            pltpu.VMEM((ept,), jnp.int32),
            pltpu.VMEM((ept,), jnp.float32),
            pltpu.VMEM((2, _L, hw), jnp.float32),
            pltpu.VMEM((2, _L, W), jnp.float32),
            pltpu.VMEM_SHARED((n_rows, W), jnp.float32),
            pltpu.SemaphoreType.DMA,
            pltpu.SemaphoreType.DMA,
            pltpu.SemaphoreType.DMA,
            pltpu.SemaphoreType.DMA,
        ],
    )
    def k(xlab_hbm, src_hbm, dst_hbm, p_hbm, z_hbm, out_hbm,
          src_v, dst_v, p_v, rows_g, rows_s, acc_sh,
          sem_g0, sem_g1, sem_s0, sem_s1):
        cid = lax.axis_index("c")
        sid = lax.axis_index("s")
        base = sid * ept
        pltpu.sync_copy(src_hbm.at[pl.ds(base, ept)], src_v)
        pltpu.sync_copy(dst_hbm.at[pl.ds(base, ept)], dst_v)
        pltpu.sync_copy(p_hbm.at[pl.ds(base, ept)], p_v)
        pltpu.sync_copy(z_hbm, acc_sh.at[pl.ds(sid * rpt, rpt)])
        plsc.subcore_barrier()
        iota = lax.iota(jnp.int32, _L)
        onehot = jnp.where(iota == 0, jnp.float32(1.0), jnp.float32(0.0))
        off = cid * n_nodes
        gsems = [sem_g0, sem_g1]
        ssems = [sem_s0, sem_s1]

        def fire_gather(c, b):
            o = c * _L
            sidx = src_v[pl.ds(o, _L)] + jnp.full((_L,), off, jnp.int32)
            pltpu.async_copy(xlab_hbm.at[sidx], rows_g.at[b], gsems[b])

        def wait_gather(b):
            pltpu.make_async_copy(xlab_hbm.at[src_v[pl.ds(0, _L)]],
                                  rows_g.at[b], gsems[b]).wait()

        def wait_scatter(b):
            pltpu.make_async_copy(rows_s.at[b],
                                  out_hbm.at[cid, pl.ds(0, _L)],
                                  ssems[b]).wait()

        def process(c, b, kk):
            o = c * _L
            wait_gather(b)

            @pl.when(kk > 0)
            def _():
                wait_scatter(b)

            pvec = p_v[pl.ds(o, _L)]
            for j in range(_L):
                pj = jnp.full((_L,), pvec[j], jnp.float32)
                for cc in range(hw // _L):
                    sl = pl.ds(cc * _L, _L)
                    rows_s[b, j, sl] = rows_g[b, j, sl] * pj
                rows_s[b, j, pl.ds(hw, _L)] = pj * onehot
            didx = dst_v[pl.ds(o, _L)]
            pltpu.async_copy(rows_s.at[b], acc_sh.at[didx], ssems[b], add=True)

        fire_gather(0, 0)

        def body(k_, carry):
            c0 = k_ * 2
            fire_gather(c0 + 1, 1)
            process(c0, 0, k_)

            @pl.when(k_ < npair - 1)
            def _():
                fire_gather(c0 + 2, 0)

            process(c0 + 1, 1, k_)
            return carry

        lax.fori_loop(0, npair, body, 0)
        wait_scatter(0)
        wait_scatter(1)
        plsc.subcore_barrier()
        pltpu.sync_copy(acc_sh.at[pl.ds(sid * rpt, rpt)],
                        out_hbm.at[cid, pl.ds(sid * rpt, rpt)])

    return k(xlab, src, dst, p, zrows)


def kernel(x, edge_index, edge_attr, W_l, W_r, att, bias, gamma, beta):
    N, D = x.shape
    E = edge_index.shape[1]
    idt = edge_index.dtype

    W = jnp.concatenate([W_l, W_r], axis=1)
    xw = _matmul(x, W)
    xl = xw[:, :D]
    xr = xw[:, D:]

    e_real = E + N
    blk = _NW * _L * 2
    e_pad = ((e_real + blk - 1) // blk) * blk
    loops = jnp.arange(N, dtype=idt)
    padz = jnp.zeros((e_pad - e_real,), dtype=idt)
    src = jnp.concatenate([edge_index[0], loops, padz]).astype(jnp.int32)
    dst = jnp.concatenate([edge_index[1], loops, padz]).astype(jnp.int32)

    p = _edge_scores(xl, xr, src, dst, att, e_real)

    hw = D // 2
    xlab = jnp.concatenate([xl[:, :hw], xl[:, hw:]], axis=0)
    n_rows = N
    zrows = jnp.zeros((n_rows // _NS, hw + _L), jnp.float32)
    accs = _scatter_pass(xlab, src, dst, p, zrows, N, n_rows, hw)

    acc = jnp.concatenate([accs[0, :N, :hw], accs[1, :N, :hw]], axis=1)
    denom = accs[0, :N, hw]

    out = _bn_epilogue(acc, denom, bias, gamma, beta)
    return (out, edge_index, edge_attr)


# speedup vs baseline: 5.6920x; 1.1475x over previous
import functools

import jax
import jax.numpy as jnp
from jax import lax
from jax.experimental import pallas as pl
from jax.experimental.pallas import tpu as pltpu
from jax.experimental.pallas import tpu_sc as plsc

_NC = 2
_NS = 16
_L = 16
_NW = _NC * _NS


def _mm_body(x_ref, w_ref, o_ref):
    o_ref[...] = jnp.dot(x_ref[...], w_ref[...],
                         preferred_element_type=jnp.float32)


def _matmul(x, w):
    M, K = x.shape
    N = w.shape[1]
    BM = 1000
    return pl.pallas_call(
        _mm_body,
        grid=(M // BM,),
        in_specs=[pl.BlockSpec((BM, K), lambda i: (i, 0)),
                  pl.BlockSpec((K, N), lambda i: (0, 0))],
        out_specs=pl.BlockSpec((BM, N), lambda i: (i, 0)),
        out_shape=jax.ShapeDtypeStruct((M, N), jnp.float32),
    )(x, w)


def _bn_body(acc_ref, den_ref, bias_ref, gamma_ref, beta_ref, o_ref):
    v = acc_ref[...] / (den_ref[...] + 1e-16) + bias_ref[...]
    n = v.shape[0]
    mean = jnp.sum(v, axis=0, keepdims=True) / n
    d = v - mean
    var = jnp.sum(d * d, axis=0, keepdims=True) / n
    o_ref[...] = gamma_ref[...] * d * jax.lax.rsqrt(var + 1e-5) + beta_ref[...]


def _bn_epilogue(acc, denom, bias, gamma, beta):
    N, D = acc.shape
    return pl.pallas_call(
        _bn_body,
        in_specs=[pl.BlockSpec((N, D), lambda: (0, 0)),
                  pl.BlockSpec((N, 1), lambda: (0, 0)),
                  pl.BlockSpec((1, D), lambda: (0, 0)),
                  pl.BlockSpec((1, D), lambda: (0, 0)),
                  pl.BlockSpec((1, D), lambda: (0, 0))],
        out_specs=pl.BlockSpec((N, D), lambda: (0, 0)),
        out_shape=jax.ShapeDtypeStruct((N, D), jnp.float32),
    )(acc, denom.reshape(N, 1), bias.reshape(1, D),
      gamma.reshape(1, D), beta.reshape(1, D))


def _edge_scores(xlr, sd, att, e_real, e_pad):
    D = xlr.shape[1]
    ept = e_pad // _NW
    nch = ept // _L
    npair = nch // 2
    mesh = plsc.VectorSubcoreMesh(core_axis_name="c", subcore_axis_name="s")

    @functools.partial(
        pl.kernel,
        out_type=jax.ShapeDtypeStruct((e_pad,), jnp.float32),
        mesh=mesh,
        compiler_params=pltpu.CompilerParams(use_tc_tiling_on_sc=False),
        scratch_types=[
            pltpu.VMEM((2 * ept,), jnp.int32),
            pltpu.VMEM((D,), jnp.float32),
            pltpu.VMEM((2, 2 * _L, D), jnp.float32),
            pltpu.VMEM((ept,), jnp.float32),
            pltpu.SemaphoreType.DMA,
            pltpu.SemaphoreType.DMA,
        ],
    )
    def k(xlr_hbm, sd_hbm, att_hbm, p_hbm,
          sd_v, att_v, rows, p_buf, sem0, sem1):
        wid = lax.axis_index("s") * _NC + lax.axis_index("c")
        base = wid * ept
        pltpu.sync_copy(sd_hbm.at[pl.ds(2 * base, 2 * ept)], sd_v)
        pltpu.sync_copy(att_hbm, att_v)
        iota = lax.iota(jnp.int32, _L)
        shufs = [jnp.bitwise_xor(iota, sh) for sh in (8, 4, 2, 1)]
        att_vecs = [att_v[pl.ds(c * _L, _L)] for c in range(D // _L)]
        sems = [sem0, sem1]

        def fire(c, b):
            pltpu.async_copy(
                xlr_hbm.at[sd_v.at[pl.ds(c * 2 * _L, 2 * _L)]],
                rows.at[b], sems[b])

        def wait(b):
            pltpu.make_async_copy(
                xlr_hbm.at[sd_v.at[pl.ds(0, 2 * _L)]],
                rows.at[b], sems[b]).wait()

        def compute(c, b):
            o = c * _L
            e = jnp.zeros((_L,), jnp.float32)
            for j in range(_L):
                acc = jnp.zeros((_L,), jnp.float32)
                for cc in range(D // _L):
                    sl = pl.ds(cc * _L, _L)
                    v = rows[b, j, sl] + rows[b, _L + j, sl]
                    acc = acc + att_vecs[cc] * jnp.where(v >= 0, v, 0.2 * v)
                for sf in shufs:
                    acc = acc + acc[sf]
                e = jnp.where(iota == j, acc, e)
            p = jnp.exp(e)
            eid = iota + jnp.full((_L,), base + o, jnp.int32)
            p = jnp.where(eid < e_real, p, 0.0)
            p_buf[pl.ds(o, _L)] = p

        fire(0, 0)

        def body(k_, carry):
            c0 = k_ * 2
            fire(c0 + 1, 1)
            wait(0)
            compute(c0, 0)

            @pl.when(k_ < npair - 1)
            def _():
                fire(c0 + 2, 0)

            wait(1)
            compute(c0 + 1, 1)
            return carry

        lax.fori_loop(0, npair, body, 0)
        pltpu.sync_copy(p_buf, p_hbm.at[pl.ds(base, ept)])

    return k(xlr, sd, att)


def _scatter_pass(xlab, src, dst, p, zrows, n_nodes, n_rows, hw):
    W = hw + _L
    e_pad = src.shape[0]
    ept = e_pad // _NS
    nch = ept // _L
    rpt = n_rows // _NS
    mesh = plsc.VectorSubcoreMesh(core_axis_name="c", subcore_axis_name="s")
    npair = nch // 2

    @functools.partial(
        pl.kernel,
        out_type=jax.ShapeDtypeStruct((_NC, n_rows, W), jnp.float32),
        mesh=mesh,
        compiler_params=pltpu.CompilerParams(use_tc_tiling_on_sc=False),
        scratch_types=[
            pltpu.VMEM((ept,), jnp.int32),
            pltpu.VMEM((ept,), jnp.int32),
            pltpu.VMEM((ept,), jnp.float32),
            pltpu.VMEM((2, _L, hw), jnp.float32),
            pltpu.VMEM((2, _L, W), jnp.float32),
            pltpu.VMEM_SHARED((n_rows, W), jnp.float32),
            pltpu.SemaphoreType.DMA,
            pltpu.SemaphoreType.DMA,
            pltpu.SemaphoreType.DMA,
            pltpu.SemaphoreType.DMA,
        ],
    )
    def k(xlab_hbm, src_hbm, dst_hbm, p_hbm, z_hbm, out_hbm,
          src_v, dst_v, p_v, rows_g, rows_s, acc_sh,
          sem_g0, sem_g1, sem_s0, sem_s1):
        cid = lax.axis_index("c")
        sid = lax.axis_index("s")
        base = sid * ept
        pltpu.sync_copy(src_hbm.at[pl.ds(base, ept)], src_v)
        pltpu.sync_copy(dst_hbm.at[pl.ds(base, ept)], dst_v)
        pltpu.sync_copy(p_hbm.at[pl.ds(base, ept)], p_v)
        pltpu.sync_copy(z_hbm, acc_sh.at[pl.ds(sid * rpt, rpt)])
        plsc.subcore_barrier()
        iota = lax.iota(jnp.int32, _L)
        onehot = jnp.where(iota == 0, jnp.float32(1.0), jnp.float32(0.0))
        off = cid * n_nodes
        gsems = [sem_g0, sem_g1]
        ssems = [sem_s0, sem_s1]

        def fire_gather(c, b):
            o = c * _L
            sidx = src_v[pl.ds(o, _L)] + jnp.full((_L,), off, jnp.int32)
            pltpu.async_copy(xlab_hbm.at[sidx], rows_g.at[b], gsems[b])

        def wait_gather(b):
            pltpu.make_async_copy(xlab_hbm.at[src_v[pl.ds(0, _L)]],
                                  rows_g.at[b], gsems[b]).wait()

        def wait_scatter(b):
            pltpu.make_async_copy(rows_s.at[b],
                                  out_hbm.at[cid, pl.ds(0, _L)],
                                  ssems[b]).wait()

        def process(c, b, kk):
            o = c * _L
            wait_gather(b)

            @pl.when(kk > 0)
            def _():
                wait_scatter(b)

            pvec = p_v[pl.ds(o, _L)]
            for j in range(_L):
                pj = jnp.full((_L,), pvec[j], jnp.float32)
                for cc in range(hw // _L):
                    sl = pl.ds(cc * _L, _L)
                    rows_s[b, j, sl] = rows_g[b, j, sl] * pj
                rows_s[b, j, pl.ds(hw, _L)] = pj * onehot
            didx = dst_v[pl.ds(o, _L)]
            pltpu.async_copy(rows_s.at[b], acc_sh.at[didx], ssems[b], add=True)

        fire_gather(0, 0)

        def body(k_, carry):
            c0 = k_ * 2
            fire_gather(c0 + 1, 1)
            process(c0, 0, k_)

            @pl.when(k_ < npair - 1)
            def _():
                fire_gather(c0 + 2, 0)

            process(c0 + 1, 1, k_)
            return carry

        lax.fori_loop(0, npair, body, 0)
        wait_scatter(0)
        wait_scatter(1)
        plsc.subcore_barrier()
        pltpu.sync_copy(acc_sh.at[pl.ds(sid * rpt, rpt)],
                        out_hbm.at[cid, pl.ds(sid * rpt, rpt)])

    return k(xlab, src, dst, p, zrows)


def kernel(x, edge_index, edge_attr, W_l, W_r, att, bias, gamma, beta):
    N, D = x.shape
    E = edge_index.shape[1]
    idt = edge_index.dtype

    W = jnp.concatenate([W_l, W_r], axis=1)
    xw = _matmul(x, W)
    xl = xw[:, :D]
    xr = xw[:, D:]
    xlr = jnp.concatenate([xl, xr], axis=0)

    e_real = E + N
    blk = _NW * _L * 2
    e_pad = ((e_real + blk - 1) // blk) * blk
    loops = jnp.arange(N, dtype=idt)
    padz = jnp.zeros((e_pad - e_real,), dtype=idt)
    src = jnp.concatenate([edge_index[0], loops, padz]).astype(jnp.int32)
    dst = jnp.concatenate([edge_index[1], loops, padz]).astype(jnp.int32)
    sd = jnp.stack([src.reshape(-1, _L), dst.reshape(-1, _L) + N],
                   axis=1).reshape(-1)

    p = _edge_scores(xlr, sd, att, e_real, e_pad)

    hw = D // 2
    xlab = jnp.concatenate([xl[:, :hw], xl[:, hw:]], axis=0)
    n_rows = N
    zrows = jnp.zeros((n_rows // _NS, hw + _L), jnp.float32)
    accs = _scatter_pass(xlab, src, dst, p, zrows, N, n_rows, hw)

    acc = jnp.concatenate([accs[0, :N, :hw], accs[1, :N, :hw]], axis=1)
    denom = accs[0, :N, hw]

    out = _bn_epilogue(acc, denom, bias, gamma, beta)
    return (out, edge_index, edge_attr)


# speedup vs baseline: 5.8442x; 1.0267x over previous
import functools

import jax
import jax.numpy as jnp
from jax import lax
from jax.experimental import pallas as pl
from jax.experimental.pallas import tpu as pltpu
from jax.experimental.pallas import tpu_sc as plsc

_NC = 2
_NS = 16
_L = 16
_NW = _NC * _NS


def _mm_body(x_ref, w_ref, o_ref):
    o_ref[...] = jnp.dot(x_ref[...], w_ref[...],
                         preferred_element_type=jnp.float32)


def _matmul_stacked(x, w):
    M, K = x.shape
    D = w.shape[1] // 2
    BM = 1000
    nb = M // BM
    return pl.pallas_call(
        _mm_body,
        grid=(2, nb),
        in_specs=[pl.BlockSpec((BM, K), lambda h, i: (i, 0)),
                  pl.BlockSpec((K, D), lambda h, i: (0, h))],
        out_specs=pl.BlockSpec((BM, D), lambda h, i: (h * nb + i, 0)),
        out_shape=jax.ShapeDtypeStruct((2 * M, D), jnp.float32),
    )(x, w)


def _bn_body(accs_ref, bias_ref, gamma_ref, beta_ref, o_ref):
    hw = o_ref.shape[1] // 2
    acc = jnp.concatenate([accs_ref[0, :, :hw], accs_ref[1, :, :hw]], axis=1)
    den = accs_ref[0, :, hw:hw + 1]
    v = acc / (den + 1e-16) + bias_ref[...]
    n = v.shape[0]
    mean = jnp.sum(v, axis=0, keepdims=True) / n
    d = v - mean
    var = jnp.sum(d * d, axis=0, keepdims=True) / n
    o_ref[...] = gamma_ref[...] * d * jax.lax.rsqrt(var + 1e-5) + beta_ref[...]


def _bn_epilogue(accs, bias, gamma, beta):
    _, N, W = accs.shape
    D = (W - _L) * 2
    return pl.pallas_call(
        _bn_body,
        in_specs=[pl.BlockSpec((2, N, W), lambda: (0, 0, 0)),
                  pl.BlockSpec((1, D), lambda: (0, 0)),
                  pl.BlockSpec((1, D), lambda: (0, 0)),
                  pl.BlockSpec((1, D), lambda: (0, 0))],
        out_specs=pl.BlockSpec((N, D), lambda: (0, 0)),
        out_shape=jax.ShapeDtypeStruct((N, D), jnp.float32),
    )(accs, bias.reshape(1, D), gamma.reshape(1, D), beta.reshape(1, D))


def _edge_scores(xlr, sd, att, e_real, e_pad):
    D = xlr.shape[1]
    ept = e_pad // _NW
    nch = ept // _L
    npair = nch // 2
    mesh = plsc.VectorSubcoreMesh(core_axis_name="c", subcore_axis_name="s")

    @functools.partial(
        pl.kernel,
        out_type=jax.ShapeDtypeStruct((e_pad,), jnp.float32),
        mesh=mesh,
        compiler_params=pltpu.CompilerParams(use_tc_tiling_on_sc=False),
        scratch_types=[
            pltpu.VMEM((2 * ept,), jnp.int32),
            pltpu.VMEM((D,), jnp.float32),
            pltpu.VMEM((2, 2 * _L, D), jnp.float32),
            pltpu.VMEM((ept,), jnp.float32),
            pltpu.SemaphoreType.DMA,
            pltpu.SemaphoreType.DMA,
        ],
    )
    def k(xlr_hbm, sd_hbm, att_hbm, p_hbm,
          sd_v, att_v, rows, p_buf, sem0, sem1):
        wid = lax.axis_index("s") * _NC + lax.axis_index("c")
        base = wid * ept
        pltpu.sync_copy(sd_hbm.at[pl.ds(2 * base, 2 * ept)], sd_v)
        pltpu.sync_copy(att_hbm, att_v)
        iota = lax.iota(jnp.int32, _L)
        shufs = [jnp.bitwise_xor(iota, sh) for sh in (8, 4, 2, 1)]
        att_vecs = [att_v[pl.ds(c * _L, _L)] for c in range(D // _L)]
        sems = [sem0, sem1]

        def fire(c, b):
            pltpu.async_copy(
                xlr_hbm.at[sd_v.at[pl.ds(c * 2 * _L, 2 * _L)]],
                rows.at[b], sems[b])

        def wait(b):
            pltpu.make_async_copy(
                xlr_hbm.at[sd_v.at[pl.ds(0, 2 * _L)]],
                rows.at[b], sems[b]).wait()

        def compute(c, b):
            o = c * _L
            e = jnp.zeros((_L,), jnp.float32)
            for j in range(_L):
                acc = jnp.zeros((_L,), jnp.float32)
                for cc in range(D // _L):
                    sl = pl.ds(cc * _L, _L)
                    v = rows[b, j, sl] + rows[b, _L + j, sl]
                    acc = acc + att_vecs[cc] * jnp.where(v >= 0, v, 0.2 * v)
                for sf in shufs:
                    acc = acc + acc[sf]
                e = jnp.where(iota == j, acc, e)
            p = jnp.exp(e)
            eid = iota + jnp.full((_L,), base + o, jnp.int32)
            p = jnp.where(eid < e_real, p, 0.0)
            p_buf[pl.ds(o, _L)] = p

        fire(0, 0)

        def body(k_, carry):
            c0 = k_ * 2
            fire(c0 + 1, 1)
            wait(0)
            compute(c0, 0)

            @pl.when(k_ < npair - 1)
            def _():
                fire(c0 + 2, 0)

            wait(1)
            compute(c0 + 1, 1)
            return carry

        lax.fori_loop(0, npair, body, 0)
        pltpu.sync_copy(p_buf, p_hbm.at[pl.ds(base, ept)])

    return k(xlr, sd, att)


def _scatter_pass(xlab, src, dst, p, zrows, n_nodes, n_rows, hw):
    W = hw + _L
    e_pad = src.shape[0]
    ept = e_pad // _NS
    nch = ept // _L
    rpt = n_rows // _NS
    mesh = plsc.VectorSubcoreMesh(core_axis_name="c", subcore_axis_name="s")
    npair = nch // 2

    @functools.partial(
        pl.kernel,
        out_type=jax.ShapeDtypeStruct((_NC, n_rows, W), jnp.float32),
        mesh=mesh,
        compiler_params=pltpu.CompilerParams(use_tc_tiling_on_sc=False),
        scratch_types=[
            pltpu.VMEM((ept,), jnp.int32),
            pltpu.VMEM((ept,), jnp.int32),
            pltpu.VMEM((ept,), jnp.float32),
            pltpu.VMEM((2, _L, hw), jnp.float32),
            pltpu.VMEM((2, _L, W), jnp.float32),
            pltpu.VMEM_SHARED((n_rows, W), jnp.float32),
            pltpu.SemaphoreType.DMA,
            pltpu.SemaphoreType.DMA,
            pltpu.SemaphoreType.DMA,
            pltpu.SemaphoreType.DMA,
        ],
    )
    def k(xlab_hbm, src_hbm, dst_hbm, p_hbm, z_hbm, out_hbm,
          src_v, dst_v, p_v, rows_g, rows_s, acc_sh,
          sem_g0, sem_g1, sem_s0, sem_s1):
        cid = lax.axis_index("c")
        sid = lax.axis_index("s")
        base = sid * ept
        pltpu.sync_copy(src_hbm.at[pl.ds(base, ept)], src_v)
        pltpu.sync_copy(dst_hbm.at[pl.ds(base, ept)], dst_v)
        pltpu.sync_copy(p_hbm.at[pl.ds(base, ept)], p_v)
        pltpu.sync_copy(z_hbm, acc_sh.at[pl.ds(sid * rpt, rpt)])
        plsc.subcore_barrier()
        iota = lax.iota(jnp.int32, _L)
        onehot = jnp.where(iota == 0, jnp.float32(1.0), jnp.float32(0.0))
        off = cid * n_nodes
        gsems = [sem_g0, sem_g1]
        ssems = [sem_s0, sem_s1]

        def fire_gather(c, b):
            o = c * _L
            sidx = src_v[pl.ds(o, _L)] + jnp.full((_L,), off, jnp.int32)
            pltpu.async_copy(xlab_hbm.at[sidx], rows_g.at[b], gsems[b])

        def wait_gather(b):
            pltpu.make_async_copy(xlab_hbm.at[src_v[pl.ds(0, _L)]],
                                  rows_g.at[b], gsems[b]).wait()

        def wait_scatter(b):
            pltpu.make_async_copy(rows_s.at[b],
                                  out_hbm.at[cid, pl.ds(0, _L)],
                                  ssems[b]).wait()

        def process(c, b, kk):
            o = c * _L
            wait_gather(b)

            @pl.when(kk > 0)
            def _():
                wait_scatter(b)

            pvec = p_v[pl.ds(o, _L)]
            for j in range(_L):
                pj = jnp.full((_L,), pvec[j], jnp.float32)
                for cc in range(hw // _L):
                    sl = pl.ds(cc * _L, _L)
                    rows_s[b, j, sl] = rows_g[b, j, sl] * pj
                rows_s[b, j, pl.ds(hw, _L)] = pj * onehot
            didx = dst_v[pl.ds(o, _L)]
            pltpu.async_copy(rows_s.at[b], acc_sh.at[didx], ssems[b], add=True)

        fire_gather(0, 0)

        def body(k_, carry):
            c0 = k_ * 2
            fire_gather(c0 + 1, 1)
            process(c0, 0, k_)

            @pl.when(k_ < npair - 1)
            def _():
                fire_gather(c0 + 2, 0)

            process(c0 + 1, 1, k_)
            return carry

        lax.fori_loop(0, npair, body, 0)
        wait_scatter(0)
        wait_scatter(1)
        plsc.subcore_barrier()
        pltpu.sync_copy(acc_sh.at[pl.ds(sid * rpt, rpt)],
                        out_hbm.at[cid, pl.ds(sid * rpt, rpt)])

    return k(xlab, src, dst, p, zrows)


def kernel(x, edge_index, edge_attr, W_l, W_r, att, bias, gamma, beta):
    N, D = x.shape
    E = edge_index.shape[1]
    idt = edge_index.dtype

    W = jnp.concatenate([W_l, W_r], axis=1)
    xlr = _matmul_stacked(x, W)
    xl = xlr[:N]

    e_real = E + N
    blk = _NW * _L * 2
    e_pad = ((e_real + blk - 1) // blk) * blk
    loops = jnp.arange(N, dtype=idt)
    padz = jnp.zeros((e_pad - e_real,), dtype=idt)
    src = jnp.concatenate([edge_index[0], loops, padz]).astype(jnp.int32)
    dst = jnp.concatenate([edge_index[1], loops, padz]).astype(jnp.int32)
    sd = jnp.stack([src.reshape(-1, _L), dst.reshape(-1, _L) + N],
                   axis=1).reshape(-1)

    p = _edge_scores(xlr, sd, att, e_real, e_pad)

    hw = D // 2
    xlab = jnp.concatenate([xl[:, :hw], xl[:, hw:]], axis=0)
    n_rows = N
    zrows = jnp.zeros((n_rows // _NS, hw + _L), jnp.float32)
    accs = _scatter_pass(xlab, src, dst, p, zrows, N, n_rows, hw)

    out = _bn_epilogue(accs, bias, gamma, beta)
    return (out, edge_index, edge_attr)
